# Initial kernel scaffold; baseline (speedup 1.0000x reference)
import functools

import jax
import jax.numpy as jnp
from jax import lax
from jax.experimental import pallas as pl
from jax.experimental.pallas import tpu as pltpu
from jax.experimental.pallas import tpu_sc as plsc

_N = 16384
_K = 8192
_D = 32

_BN = 1024
_KC = 2048


def _argmin_body(z_ref, c_ref, idx_ref):
    z = z_ref[...]
    zz = jnp.sum(z * z, axis=1, keepdims=True)
    best_val = jnp.full((_BN, 128), jnp.inf, jnp.float32)
    best_idx = jnp.zeros((_BN, 128), jnp.int32)
    lane = lax.broadcasted_iota(jnp.int32, (_BN, 128), 1)
    for kc in range(0, _K, _KC):
        c = c_ref[kc:kc + _KC, :]
        cc = jnp.sum(c * c, axis=1)
        mm = lax.dot_general(z, c, (((1,), (1,)), ((), ())),
                             preferred_element_type=jnp.float32)
        d = zz - 2.0 * mm + cc[None, :]
        for s in range(0, _KC, 128):
            strip = d[:, s:s + 128]
            sidx = lane + (kc + s)
            m = strip < best_val
            best_idx = jnp.where(m, sidx, best_idx)
            best_val = jnp.where(m, strip, best_val)
    row_min = jnp.min(best_val, axis=1, keepdims=True)
    cand = jnp.where(best_val == row_min, best_idx, _K)
    idx_ref[...] = jnp.min(cand, axis=1)


def _make_argmin_call(interpret=False):
    return pl.pallas_call(
        _argmin_body,
        grid=(_N // _BN,),
        in_specs=[
            pl.BlockSpec((_BN, _D), lambda i: (i, 0)),
            pl.BlockSpec((_K, _D), lambda i: (0, 0)),
        ],
        out_specs=pl.BlockSpec((_BN,), lambda i: (i,)),
        out_shape=jax.ShapeDtypeStruct((_N,), jnp.int32),
        interpret=interpret,
    )


def _gather_call(codebook, indices):
    info = plsc.get_sparse_core_info()
    nw = info.num_cores * info.num_subcores
    bpw = _N // nw
    mesh = plsc.VectorSubcoreMesh(core_axis_name="c", subcore_axis_name="s")

    @functools.partial(
        pl.kernel, mesh=mesh,
        out_type=jax.ShapeDtypeStruct((_N, _D), jnp.float32),
        scratch_types=[
            pltpu.VMEM((bpw,), jnp.int32),
            pltpu.VMEM((bpw, _D), jnp.float32),
            pltpu.SemaphoreType.DMA,
        ],
    )
    def k(table_hbm, idx_hbm, out_hbm, idx_v, rows_v, sem):
        wid = lax.axis_index("s") * info.num_cores + lax.axis_index("c")
        base = wid * bpw
        pltpu.sync_copy(idx_hbm.at[pl.ds(base, bpw)], idx_v)
        pltpu.async_copy(table_hbm.at[idx_v], rows_v, sem).wait()
        pltpu.sync_copy(rows_v, out_hbm.at[pl.ds(base, bpw)])

    return k(codebook, indices)


def kernel(z_e_x, codebook):
    idx = _make_argmin_call()(z_e_x, codebook)
    z_q = _gather_call(codebook, idx)
    return (z_q, z_q)


# speedup vs baseline: 1.4690x; 1.4690x over previous
import functools

import jax
import jax.numpy as jnp
from jax import lax
from jax.experimental import pallas as pl
from jax.experimental.pallas import tpu as pltpu
from jax.experimental.pallas import tpu_sc as plsc

_N = 16384
_K = 8192
_D = 32

_BN = 512
_KC = 4096


def _bf16_round(x):
    return x.astype(jnp.bfloat16).astype(jnp.float32)


def _argmin_body(z_ref, c_ref, zz_ref, idx_ref):
    z = z_ref[...]
    zz = zz_ref[...][:, None]
    lane = lax.broadcasted_iota(jnp.int32, (_BN, 128), 1)
    acc = None
    idx = None
    for kc in range(0, _K, _KC):
        c = c_ref[kc:kc + _KC, :]
        cc = jnp.sum(c * c, axis=1)
        mm = lax.dot_general(z, c, (((1,), (1,)), ((), ())),
                             preferred_element_type=jnp.float32)
        d = zz - 2.0 * mm + cc[None, :]
        best_val = jnp.full((_BN, 128), jnp.inf, jnp.float32)
        best_idx = jnp.zeros((_BN, 128), jnp.int32)
        for s in range(0, _KC, 128):
            strip = d[:, s:s + 128]
            sidx = lane + (kc + s)
            m = strip < best_val
            best_idx = jnp.where(m, sidx, best_idx)
            best_val = jnp.where(m, strip, best_val)
        w = jnp.min(best_val, axis=1)
        j = jnp.min(jnp.where(best_val == w[:, None], best_idx, _K), axis=1)
        if acc is None:
            acc = _bf16_round(w)
            idx = j
        else:
            take = w < acc
            idx = jnp.where(take, j, idx)
            acc = jnp.where(take, _bf16_round(w), acc)
    idx_ref[...] = idx


def _make_argmin_call(interpret=False):
    return pl.pallas_call(
        _argmin_body,
        grid=(_N // _BN,),
        in_specs=[
            pl.BlockSpec((_BN, _D), lambda i: (i, 0)),
            pl.BlockSpec((_K, _D), lambda i: (0, 0)),
            pl.BlockSpec((_BN,), lambda i: (i,)),
        ],
        out_specs=pl.BlockSpec((_BN,), lambda i: (i,)),
        out_shape=jax.ShapeDtypeStruct((_N,), jnp.int32),
        interpret=interpret,
    )


def _gather_call(codebook, indices):
    info = plsc.get_sparse_core_info()
    nw = info.num_cores * info.num_subcores
    bpw = _N // nw
    mesh = plsc.VectorSubcoreMesh(core_axis_name="c", subcore_axis_name="s")

    @functools.partial(
        pl.kernel, mesh=mesh,
        out_type=jax.ShapeDtypeStruct((_N, _D), jnp.float32),
        compiler_params=pltpu.CompilerParams(use_tc_tiling_on_sc=False),
        scratch_types=[
            pltpu.VMEM((bpw,), jnp.int32),
            pltpu.VMEM((bpw, _D), jnp.float32),
            pltpu.SemaphoreType.DMA,
        ],
    )
    def k(table_hbm, idx_hbm, out_hbm, idx_v, rows_v, sem):
        wid = lax.axis_index("s") * info.num_cores + lax.axis_index("c")
        base = wid * bpw
        pltpu.sync_copy(idx_hbm.at[pl.ds(base, bpw)], idx_v)
        pltpu.async_copy(table_hbm.at[idx_v], rows_v, sem).wait()
        pltpu.sync_copy(rows_v, out_hbm.at[pl.ds(base, bpw)])

    return k(codebook, indices)


def kernel(z_e_x, codebook):
    zz = jnp.sum(z_e_x * z_e_x, axis=1)
    idx = _make_argmin_call()(z_e_x, codebook, zz)
    z_q = _gather_call(codebook, idx)
    return (z_q, z_q)
